# Initial kernel scaffold; baseline (speedup 1.0000x reference)
import jax
import jax.numpy as jnp
from jax.experimental import pallas as pl


def kernel(boxes, scores):
    raise NotImplementedError("write your pallas kernel here")


# speedup vs baseline: 4.7634x; 4.7634x over previous
import jax
import jax.numpy as jnp
from jax import lax
from jax.experimental import pallas as pl
from jax.experimental.pallas import tpu as pltpu

_N = 20000
_K = 80
_M = 4096
_TOPK = 100
_IMG_W = 1333.0
_IMG_H = 800.0
_SCORE_THRESH = 0.05
_NMS_THRESH = 0.5
_OFFSET = 1334.0

_CH = 32
_NR = _M // 128


def _row_to_mat(x):
    return jnp.concatenate([x[:, 128 * a:128 * (a + 1)] for a in range(_NR)],
                           axis=0)


def _mat_to_row(x):
    return jnp.concatenate([x[a:a + 1, :] for a in range(_NR)], axis=1)


def _nms_body(cbT_ref, cbraw_ref, clsrow_ref, clscol_ref, srow_ref, scol_ref,
              out_ref, o_ref):
    f32 = jnp.float32
    i32 = jnp.int32

    offr = clsrow_ref[...] * _OFFSET
    x1r = jnp.clip(cbT_ref[pl.ds(0, 1), :], 0.0, _IMG_W) + offr
    y1r = jnp.clip(cbT_ref[pl.ds(1, 1), :], 0.0, _IMG_H) + offr
    x2r = jnp.clip(cbT_ref[pl.ds(2, 1), :], 0.0, _IMG_W) + offr
    y2r = jnp.clip(cbT_ref[pl.ds(3, 1), :], 0.0, _IMG_H) + offr
    arear = jnp.maximum(x2r - x1r, 0.0) * jnp.maximum(y2r - y1r, 0.0)

    icol = lax.broadcasted_iota(i32, (_CH, _M), 1)
    jrow = lax.broadcasted_iota(i32, (_CH, _M), 0)

    def build(c, _):
        offc = clscol_ref[pl.ds(c * _CH, _CH), pl.ds(0, 1)] * _OFFSET
        x1c = jnp.clip(cbraw_ref[pl.ds(c * _CH, _CH), pl.ds(0, 1)], 0.0, _IMG_W) + offc
        y1c = jnp.clip(cbraw_ref[pl.ds(c * _CH, _CH), pl.ds(1, 1)], 0.0, _IMG_H) + offc
        x2c = jnp.clip(cbraw_ref[pl.ds(c * _CH, _CH), pl.ds(2, 1)], 0.0, _IMG_W) + offc
        y2c = jnp.clip(cbraw_ref[pl.ds(c * _CH, _CH), pl.ds(3, 1)], 0.0, _IMG_H) + offc
        areac = jnp.maximum(x2c - x1c, 0.0) * jnp.maximum(y2c - y1c, 0.0)

        xx1 = jnp.maximum(x1c, x1r)
        yy1 = jnp.maximum(y1c, y1r)
        xx2 = jnp.minimum(x2c, x2r)
        yy2 = jnp.minimum(y2c, y2r)
        inter = jnp.maximum(xx2 - xx1, 0.0) * jnp.maximum(yy2 - yy1, 0.0)
        union = areac + arear - inter
        iou = inter / jnp.maximum(union, 1e-9)
        over = (iou > _NMS_THRESH) & ((c * _CH + jrow) < icol)
        o_ref[pl.ds(c * _CH, _CH), :] = jnp.where(over, 1.0, 0.0).astype(jnp.bfloat16)
        return 0

    lax.fori_loop(0, _M // _CH, build, 0)

    valid = jnp.where(srow_ref[...] > _SCORE_THRESH, 1.0, 0.0).astype(f32)

    def cond(carry):
        _, changed, it = carry
        return changed & (it < _M)

    def body(carry):
        kF, _, it = carry
        kB = kF.astype(jnp.bfloat16)

        s = jnp.zeros((1, _M), f32)
        for c in range(_NR):
            s = s + lax.dot_general(
                kB[:, 128 * c:128 * (c + 1)], o_ref[pl.ds(128 * c, 128), :],
                (((1,), (0,)), ((), ())), preferred_element_type=f32)
        newk = jnp.where((valid > 0.5) & (s < 0.5), 1.0, 0.0).astype(f32)
        changed = jnp.max(jnp.abs(newk - kF)) > 0.0
        return newk, changed, it + 1

    kF, _, _ = lax.while_loop(cond, body,
                              (valid, jnp.array(True), jnp.array(0, i32)))

    k2 = _row_to_mat(kF)
    c128 = k2
    sh = 1
    while sh < 128:
        c128 = c128 + jnp.concatenate(
            [jnp.zeros((_NR, sh), f32), c128[:, : 128 - sh]], axis=1)
        sh *= 2
    rowsum = jnp.sum(k2, axis=1, keepdims=True)
    inc = rowsum
    sh = 1
    while sh < _NR:
        inc = inc + jnp.concatenate(
            [jnp.zeros((sh, 1), f32), inc[: _NR - sh, :]], axis=0)
        sh *= 2
    r2 = c128 + (inc - rowsum)
    r_row = _mat_to_row(r2)

    srank = lax.broadcasted_iota(i32, (128, _M), 0)
    sel = jnp.where((kF > 0.5) & (r_row.astype(i32) == srank + 1),
                    1.0, 0.0).astype(f32)

    c0 = jnp.clip(cbraw_ref[:, pl.ds(0, 1)], 0.0, _IMG_W)
    c1 = jnp.clip(cbraw_ref[:, pl.ds(1, 1)], 0.0, _IMG_H)
    c2 = jnp.clip(cbraw_ref[:, pl.ds(2, 1)], 0.0, _IMG_W)
    c3 = jnp.clip(cbraw_ref[:, pl.ds(3, 1)], 0.0, _IMG_H)
    dmat = jnp.concatenate(
        [c0, c1, c2, c3, scol_ref[...], jnp.zeros((_M, 123), f32)], axis=1)
    out_ref[...] = lax.dot_general(sel, dmat, (((1,), (0,)), ((), ())),
                                   preferred_element_type=f32)


def _nms_select(cbT, cbraw, cls_row, cls_col, s_row, s_col):
    return pl.pallas_call(
        _nms_body,
        out_shape=jax.ShapeDtypeStruct((128, 128), jnp.float32),
        scratch_shapes=[pltpu.VMEM((_M, _M), jnp.bfloat16)],
        compiler_params=pltpu.CompilerParams(
            vmem_limit_bytes=100 * 1024 * 1024),
    )(cbT, cbraw, cls_row, cls_col, s_row, s_col)


def kernel(boxes, scores):
    sfg = scores[:, :-1]
    masked = jnp.where(sfg > _SCORE_THRESH, sfg, -1.0)
    cand_scores, cand_idx = lax.top_k(masked.reshape(-1), _M)
    box_idx = cand_idx // _K
    cls = (cand_idx % _K).astype(jnp.float32)
    cbraw = jnp.take(boxes, box_idx, axis=0)

    out = _nms_select(
        cbraw.T,
        cbraw,
        cls.reshape(1, _M),
        cls.reshape(_M, 1),
        cand_scores.reshape(1, _M),
        cand_scores.reshape(_M, 1),
    )
    return out[:_TOPK, :5]


# speedup vs baseline: 4.7637x; 1.0001x over previous
import jax
import jax.numpy as jnp
from jax import lax
from jax.experimental import pallas as pl
from jax.experimental.pallas import tpu as pltpu

_N = 20000
_K = 80
_M = 4096
_TOPK = 100
_IMG_W = 1333.0
_IMG_H = 800.0
_SCORE_THRESH = 0.05
_NMS_THRESH = 0.5
_OFFSET = 1334.0

_CH = 32
_NR = _M // 128


def _row_to_mat(x):
    return jnp.concatenate([x[:, 128 * a:128 * (a + 1)] for a in range(_NR)],
                           axis=0)


def _mat_to_row(x):
    return jnp.concatenate([x[a:a + 1, :] for a in range(_NR)], axis=1)


def _nms_body(cbT_ref, cbraw_ref, clsrow_ref, clscol_ref, srow_ref, scol_ref,
              out_ref, o_ref):
    f32 = jnp.float32
    i32 = jnp.int32

    offr = clsrow_ref[...] * _OFFSET
    x1r = jnp.clip(cbT_ref[pl.ds(0, 1), :], 0.0, _IMG_W) + offr
    y1r = jnp.clip(cbT_ref[pl.ds(1, 1), :], 0.0, _IMG_H) + offr
    x2r = jnp.clip(cbT_ref[pl.ds(2, 1), :], 0.0, _IMG_W) + offr
    y2r = jnp.clip(cbT_ref[pl.ds(3, 1), :], 0.0, _IMG_H) + offr
    arear = jnp.maximum(x2r - x1r, 0.0) * jnp.maximum(y2r - y1r, 0.0)

    icol = lax.broadcasted_iota(i32, (_CH, _M), 1)
    jrow = lax.broadcasted_iota(i32, (_CH, _M), 0)

    def build(c, _):
        offc = clscol_ref[pl.ds(c * _CH, _CH), pl.ds(0, 1)] * _OFFSET
        x1c = jnp.clip(cbraw_ref[pl.ds(c * _CH, _CH), pl.ds(0, 1)], 0.0, _IMG_W) + offc
        y1c = jnp.clip(cbraw_ref[pl.ds(c * _CH, _CH), pl.ds(1, 1)], 0.0, _IMG_H) + offc
        x2c = jnp.clip(cbraw_ref[pl.ds(c * _CH, _CH), pl.ds(2, 1)], 0.0, _IMG_W) + offc
        y2c = jnp.clip(cbraw_ref[pl.ds(c * _CH, _CH), pl.ds(3, 1)], 0.0, _IMG_H) + offc
        areac = jnp.maximum(x2c - x1c, 0.0) * jnp.maximum(y2c - y1c, 0.0)

        xx1 = jnp.maximum(x1c, x1r)
        yy1 = jnp.maximum(y1c, y1r)
        xx2 = jnp.minimum(x2c, x2r)
        yy2 = jnp.minimum(y2c, y2r)
        inter = jnp.maximum(xx2 - xx1, 0.0) * jnp.maximum(yy2 - yy1, 0.0)
        union = areac + arear - inter
        iou = inter / jnp.maximum(union, 1e-9)
        over = (iou > _NMS_THRESH) & ((c * _CH + jrow) < icol)
        o_ref[pl.ds(c * _CH, _CH), :] = jnp.where(over, 1.0, 0.0).astype(jnp.bfloat16)
        return 0

    lax.fori_loop(0, _M // _CH, build, 0)

    valid = jnp.where(srow_ref[...] > _SCORE_THRESH, 1.0, 0.0).astype(f32)

    def cond(carry):
        _, changed, it = carry
        return changed & (it < _M)

    def body(carry):
        kF, _, it = carry
        kB = kF.astype(jnp.bfloat16)

        s = jnp.zeros((1, _M), f32)
        for c in range(_NR):
            s = s + lax.dot_general(
                kB[:, 128 * c:128 * (c + 1)], o_ref[pl.ds(128 * c, 128), :],
                (((1,), (0,)), ((), ())), preferred_element_type=f32)
        newk = jnp.where((valid > 0.5) & (s < 0.5), 1.0, 0.0).astype(f32)
        changed = jnp.max(jnp.abs(newk - kF)) > 0.0
        return newk, changed, it + 1

    kF, _, _ = lax.while_loop(cond, body,
                              (valid, jnp.array(True), jnp.array(0, i32)))

    k2 = _row_to_mat(kF)
    c128 = k2
    sh = 1
    while sh < 128:
        c128 = c128 + jnp.concatenate(
            [jnp.zeros((_NR, sh), f32), c128[:, : 128 - sh]], axis=1)
        sh *= 2
    rowsum = jnp.sum(k2, axis=1, keepdims=True)
    inc = rowsum
    sh = 1
    while sh < _NR:
        inc = inc + jnp.concatenate(
            [jnp.zeros((sh, 1), f32), inc[: _NR - sh, :]], axis=0)
        sh *= 2
    r2 = c128 + (inc - rowsum)
    r_row = _mat_to_row(r2)

    srank = lax.broadcasted_iota(i32, (128, _M), 0)
    sel = jnp.where((kF > 0.5) & (r_row.astype(i32) == srank + 1),
                    1.0, 0.0).astype(f32)

    nx1 = jnp.clip(cbT_ref[pl.ds(0, 1), :], 0.0, _IMG_W)
    ny1 = jnp.clip(cbT_ref[pl.ds(1, 1), :], 0.0, _IMG_H)
    nx2 = jnp.clip(cbT_ref[pl.ds(2, 1), :], 0.0, _IMG_W)
    ny2 = jnp.clip(cbT_ref[pl.ds(3, 1), :], 0.0, _IMG_H)
    cols = []
    for val in (nx1, ny1, nx2, ny2, srow_ref[...]):
        cols.append(jnp.sum(sel * val, axis=1, keepdims=True))
    out_ref[...] = jnp.concatenate(cols + [jnp.zeros((128, 123), f32)], axis=1)


def _nms_select(cbT, cbraw, cls_row, cls_col, s_row, s_col):
    return pl.pallas_call(
        _nms_body,
        out_shape=jax.ShapeDtypeStruct((128, 128), jnp.float32),
        scratch_shapes=[pltpu.VMEM((_M, _M), jnp.bfloat16)],
        compiler_params=pltpu.CompilerParams(
            vmem_limit_bytes=100 * 1024 * 1024),
    )(cbT, cbraw, cls_row, cls_col, s_row, s_col)


def kernel(boxes, scores):
    sfg = scores[:, :-1]
    masked = jnp.where(sfg > _SCORE_THRESH, sfg, -1.0)
    cand_scores, cand_idx = lax.top_k(masked.reshape(-1), _M)
    box_idx = cand_idx // _K
    cls = (cand_idx % _K).astype(jnp.float32)
    cbraw = jnp.take(boxes, box_idx, axis=0)

    out = _nms_select(
        cbraw.T,
        cbraw,
        cls.reshape(1, _M),
        cls.reshape(_M, 1),
        cand_scores.reshape(1, _M),
        cand_scores.reshape(_M, 1),
    )
    return out[:_TOPK, :5]
